# baseline (speedup 1.0000x reference)
import functools

import jax
import jax.numpy as jnp
from jax import lax
from jax.experimental import pallas as pl
from jax.experimental.pallas import tpu as pltpu
from jax.experimental.pallas import tpu_sc as plsc

_NUM_STYLES = 1000
_STYLE_DIM = 64
_PAD_DIM = 128
_BATCH = 16384

_B_SC = _BATCH // 2
_B_TC = _BATCH - _B_SC

_NC = 2
_NS = 16
_NW = _NC * _NS
_B_PER_W = _B_SC // _NW
_NCHUNK = 4
_B_CHUNK = _B_PER_W // _NCHUNK

_TC_BLK = 512
_NB_TC = _B_TC // _TC_BLK
_K = 1024

_mesh = plsc.VectorSubcoreMesh(core_axis_name="c", subcore_axis_name="s")


@functools.partial(
    pl.kernel,
    mesh=_mesh,
    out_type=jax.ShapeDtypeStruct((_B_SC, _PAD_DIM), jnp.float32),
    scratch_types=[
        pltpu.VMEM((_B_PER_W,), jnp.int32),
        pltpu.VMEM((_B_PER_W, _PAD_DIM), jnp.float32),
        pltpu.VMEM_SHARED((_NUM_STYLES, _PAD_DIM), jnp.float32),
        [pltpu.SemaphoreType.DMA] * _NCHUNK,
        [pltpu.SemaphoreType.DMA] * _NCHUNK,
    ],
)
def _sc_gather(table_hbm, idx_hbm, out_hbm, idx_v, rows_v, table_sp,
               gsems, wsems):
    sid = lax.axis_index("s")
    wid = sid * _NC + lax.axis_index("c")
    base = wid * _B_PER_W

    @pl.when(sid == 0)
    def _stage():
        pltpu.sync_copy(table_hbm, table_sp)

    pltpu.sync_copy(idx_hbm.at[pl.ds(base, _B_PER_W)], idx_v)
    plsc.subcore_barrier()
    gathers = []
    for k in range(_NCHUNK):
        gathers.append(
            pltpu.async_copy(
                table_sp.at[idx_v.at[pl.ds(k * _B_CHUNK, _B_CHUNK)]],
                rows_v.at[pl.ds(k * _B_CHUNK, _B_CHUNK)],
                gsems[k],
            )
        )
    writes = []
    for k in range(_NCHUNK):
        gathers[k].wait()
        writes.append(
            pltpu.async_copy(
                rows_v.at[pl.ds(k * _B_CHUNK, _B_CHUNK)],
                out_hbm.at[pl.ds(base + k * _B_CHUNK, _B_CHUNK)],
                wsems[k],
            )
        )
    for k in range(_NCHUNK):
        writes[k].wait()


def _tc_body(idx_ref, hi_ref, mid_ref, lo_ref, out_ref):
    idx = idx_ref[0]
    iota = lax.broadcasted_iota(jnp.int32, (_K, _TC_BLK), 0)
    oh = jnp.where(iota == idx, 1.0, 0.0).astype(jnp.bfloat16)
    dn = (((0,), (0,)), ((), ()))
    acc = lax.dot_general(oh, hi_ref[...], dn,
                          preferred_element_type=jnp.float32)
    acc += lax.dot_general(oh, mid_ref[...], dn,
                           preferred_element_type=jnp.float32)
    acc += lax.dot_general(oh, lo_ref[...], dn,
                           preferred_element_type=jnp.float32)
    out_ref[...] = acc


_tc_gather = pl.pallas_call(
    _tc_body,
    grid=(_NB_TC,),
    in_specs=[
        pl.BlockSpec((1, 1, _TC_BLK), lambda i: (i, 0, 0)),
        pl.BlockSpec((_K, _STYLE_DIM), lambda i: (0, 0)),
        pl.BlockSpec((_K, _STYLE_DIM), lambda i: (0, 0)),
        pl.BlockSpec((_K, _STYLE_DIM), lambda i: (0, 0)),
    ],
    out_specs=pl.BlockSpec((_TC_BLK, _STYLE_DIM), lambda i: (i, 0)),
    out_shape=jax.ShapeDtypeStruct((_B_TC, _STYLE_DIM), jnp.float32),
)


def kernel(style_id, embed_weight):
    idx = style_id.astype(jnp.int32)
    table128 = jnp.pad(embed_weight, ((0, 0), (0, _PAD_DIM - _STYLE_DIM)))

    hi = embed_weight.astype(jnp.bfloat16)
    r1 = embed_weight - hi.astype(jnp.float32)
    mid = r1.astype(jnp.bfloat16)
    lo = (r1 - mid.astype(jnp.float32)).astype(jnp.bfloat16)
    pad_k = ((0, _K - _NUM_STYLES), (0, 0))
    hi = jnp.pad(hi, pad_k)
    mid = jnp.pad(mid, pad_k)
    lo = jnp.pad(lo, pad_k)

    sc128 = _sc_gather(table128, idx[:_B_SC])
    tc_out = _tc_gather(idx[_B_SC:].reshape(_NB_TC, 1, _TC_BLK), hi, mid, lo)
    return jnp.concatenate([sc128[:, :_STYLE_DIM], tc_out], axis=0)


# speedup vs baseline: 1.7460x; 1.7460x over previous
import functools

import jax
import jax.numpy as jnp
from jax import lax
from jax.experimental import pallas as pl
from jax.experimental.pallas import tpu as pltpu
from jax.experimental.pallas import tpu_sc as plsc

_NUM_STYLES = 1000
_STYLE_DIM = 64
_PAD_DIM = 128
_BATCH = 16384

_NC = 2
_NS = 16
_NW = _NC * _NS
_B_PER_W = _BATCH // _NW
_NCHUNK = 4
_B_CHUNK = _B_PER_W // _NCHUNK

_mesh = plsc.VectorSubcoreMesh(core_axis_name="c", subcore_axis_name="s")


@functools.partial(
    pl.kernel,
    mesh=_mesh,
    out_type=jax.ShapeDtypeStruct((_BATCH, _PAD_DIM), jnp.float32),
    scratch_types=[
        pltpu.VMEM((_B_PER_W,), jnp.int32),
        pltpu.VMEM((_B_PER_W, _PAD_DIM), jnp.float32),
        pltpu.VMEM_SHARED((_NUM_STYLES, _PAD_DIM), jnp.float32),
        [pltpu.SemaphoreType.DMA] * _NCHUNK,
        [pltpu.SemaphoreType.DMA] * _NCHUNK,
    ],
)
def _gather_kernel(table_hbm, idx_hbm, out_hbm, idx_v, rows_v, table_sp,
                   gsems, wsems):
    sid = lax.axis_index("s")
    wid = sid * _NC + lax.axis_index("c")
    base = wid * _B_PER_W

    @pl.when(sid == 0)
    def _stage():
        pltpu.sync_copy(table_hbm, table_sp)

    pltpu.sync_copy(idx_hbm.at[pl.ds(base, _B_PER_W)], idx_v)
    plsc.subcore_barrier()
    gathers = []
    for k in range(_NCHUNK):
        gathers.append(
            pltpu.async_copy(
                table_sp.at[idx_v.at[pl.ds(k * _B_CHUNK, _B_CHUNK)]],
                rows_v.at[pl.ds(k * _B_CHUNK, _B_CHUNK)],
                gsems[k],
            )
        )
    writes = []
    for k in range(_NCHUNK):
        gathers[k].wait()
        writes.append(
            pltpu.async_copy(
                rows_v.at[pl.ds(k * _B_CHUNK, _B_CHUNK)],
                out_hbm.at[pl.ds(base + k * _B_CHUNK, _B_CHUNK)],
                wsems[k],
            )
        )
    for k in range(_NCHUNK):
        writes[k].wait()


def kernel(style_id, embed_weight):
    table128 = jnp.pad(embed_weight, ((0, 0), (0, _PAD_DIM - _STYLE_DIM)))
    out128 = _gather_kernel(table128, style_id.astype(jnp.int32))
    return out128[:, :_STYLE_DIM]
